# Initial kernel scaffold; baseline (speedup 1.0000x reference)
import jax
import jax.numpy as jnp
from jax.experimental import pallas as pl


def kernel(x, edge_index, W, att_src, att_dst, b_conv, W1, b1, W2, b2, gamma, beta):
    raise NotImplementedError("write your pallas kernel here")


# speedup vs baseline: 11.2478x; 11.2478x over previous
import functools

import jax
import jax.numpy as jnp
from jax import lax
from jax.experimental import pallas as pl
from jax.experimental.pallas import tpu as pltpu
from jax.experimental.pallas import tpu_sc as plsc

N = 10000
IN = 256
H = 4
C = 256
HC = H * C
DFF = 512
G = 8
GW = 128
NPAD = 10240
ROWS_PER_TILE = NPAD // 16
E_BASE = 160000
E2 = E_BASE + N
PER_TILE = 10752
EPAD = 16 * PER_TILE
TB = 3584
NSUP = TB // 64
NBLK = 3

_f32 = jnp.float32
_i32 = jnp.int32


def _stage_a_body(x_ref, w_ref, s_ref, xp_ref, aall_ref):
    xp = jnp.dot(x_ref[...], w_ref[...], preferred_element_type=_f32)
    aall_ref[...] = jnp.dot(xp, s_ref[...], preferred_element_type=_f32)
    for g in range(G):
        xp_ref[g] = xp[:, g * GW:(g + 1) * GW]


def _stage_a(xpad, W, S):
    blk = 640
    grid = NPAD // blk
    return pl.pallas_call(
        _stage_a_body,
        grid=(grid,),
        in_specs=[
            pl.BlockSpec((blk, IN), lambda i: (i, 0)),
            pl.BlockSpec((IN, HC), lambda i: (0, 0)),
            pl.BlockSpec((HC, 2 * H), lambda i: (0, 0)),
        ],
        out_specs=[
            pl.BlockSpec((G, blk, GW), lambda i: (0, i, 0)),
            pl.BlockSpec((blk, 2 * H), lambda i: (i, 0)),
        ],
        out_shape=[
            jax.ShapeDtypeStruct((G, NPAD, GW), _f32),
            jax.ShapeDtypeStruct((NPAD, 2 * H), _f32),
        ],
    )(xpad, W, S)


def _stage_b_body(aall_hbm, src_hbm, dst_hbm, dst2_hbm,
                  ex_hbm, rden_hbm,
                  table_v, src_v, dst_v, denacc, ex_st, tmp_v, colbuf,
                  slab_sh):
    c = lax.axis_index("c")
    s = lax.axis_index("s")
    zeros16 = jnp.zeros((16,), _f32)

    @pl.when(c == 0)
    def _():
        pltpu.sync_copy(aall_hbm, table_v)

        for h in range(H):
            def zd(i, _):
                denacc[pl.ds(i * 16, 16)] = zeros16
                return 0
            lax.fori_loop(0, NPAD // 16, zd, 0)

            base_e = s * PER_TILE
            for blk in range(NBLK):
                off = pl.multiple_of(base_e + blk * TB, 512)
                pltpu.sync_copy(src_hbm.at[pl.ds(off, TB)], src_v)
                pltpu.sync_copy(dst_hbm.at[pl.ds(off, TB)], dst_v)

                def chunk_body(q, _):
                    didx = dst_v[pl.ds(q * 16, 16)]
                    sidx = src_v[pl.ds(q * 16, 16)]
                    sv = plsc.load_gather(table_v, [sidx * 8 + h])
                    dv = plsc.load_gather(table_v, [didx * 8 + (H + h)])
                    al = sv + dv
                    al = jnp.where(al >= 0.0, al, al * jnp.float32(0.2))
                    exv = jnp.exp(al)
                    ex_st[pl.ds(q * 16, 16)] = exv
                    plsc.addupdate_scatter(denacc, [didx], exv)
                    return 0

                lax.fori_loop(0, TB // 16, chunk_body, 0)
                pltpu.sync_copy(
                    ex_st,
                    ex_hbm.at[pl.ds(pl.multiple_of(h * EPAD + off, 512),
                                    TB)])

            pltpu.sync_copy(denacc, slab_sh.at[s])
            plsc.subcore_barrier()
            my0 = pl.multiple_of(s * ROWS_PER_TILE, 128)
            for t in range(16):
                pltpu.sync_copy(slab_sh.at[t, pl.ds(my0, ROWS_PER_TILE)],
                                tmp_v)
                def acc_body(j, _):
                    vv = tmp_v[pl.ds(j * 16, 16)]
                    if t == 0:
                        colbuf[pl.ds(j * 16, 16)] = vv
                    else:
                        colbuf[pl.ds(j * 16, 16)] = (
                            colbuf[pl.ds(j * 16, 16)] + vv)
                    return 0
                lax.fori_loop(0, ROWS_PER_TILE // 16, acc_body, 0)
            def rec_body(j, _):
                vv = colbuf[pl.ds(j * 16, 16)]
                colbuf[pl.ds(j * 16, 16)] = 1.0 / (vv + jnp.float32(1e-16))
                return 0
            lax.fori_loop(0, ROWS_PER_TILE // 16, rec_body, 0)
            pltpu.sync_copy(
                colbuf,
                rden_hbm.at[pl.ds(
                    pl.multiple_of(h * NPAD + s * ROWS_PER_TILE, 128),
                    ROWS_PER_TILE)])
            plsc.subcore_barrier()


def _stage_b(aall_flat, src, dst, dst2):
    mesh = plsc.VectorSubcoreMesh(core_axis_name="c", subcore_axis_name="s")
    return pl.kernel(
        _stage_b_body,
        out_type=(jax.ShapeDtypeStruct((H * EPAD,), _f32),
                  jax.ShapeDtypeStruct((H * NPAD,), _f32)),
        mesh=mesh,
        compiler_params=pltpu.CompilerParams(needs_layout_passes=False),
        scratch_types=[
            pltpu.VMEM((NPAD * 8,), _f32),
            pltpu.VMEM((TB,), _i32),
            pltpu.VMEM((TB,), _i32),
            pltpu.VMEM((NPAD,), _f32),
            pltpu.VMEM((TB,), _f32),
            pltpu.VMEM((ROWS_PER_TILE,), _f32),
            pltpu.VMEM((ROWS_PER_TILE,), _f32),
            pltpu.MemorySpace.VMEM_SHARED((16, NPAD), _f32),
        ],
    )(aall_flat, src, dst, dst2)


def _stage_c_body(xp_hbm, src_hbm, dst_hbm, dst2_hbm, ex_hbm, rden_hbm,
                  out_hbm,
                  rden_v, src_v, dst_v, dst2_v, ex_v, gidx_v, abuf, stage_v,
                  zbuf, acc_sh, gsem):
    c = lax.axis_index("c")
    s = lax.axis_index("s")
    iota16 = lax.iota(_i32, 16)
    zeros16 = jnp.zeros((16,), _f32)

    def zb(i, _):
        r = i // 8
        jj = (i % 8) * 16
        zbuf[r, pl.ds(jj, 16)] = zeros16
        return 0
    lax.fori_loop(0, 64 * 8, zb, 0)

    for g4 in range(4):
        g = c * 4 + g4
        head = (c * 4 + g4) // 2
        pltpu.sync_copy(
            rden_hbm.at[pl.ds(pl.multiple_of(head * NPAD, 128), NPAD)],
            rden_v)
        for k in range(ROWS_PER_TILE // 64):
            pltpu.sync_copy(
                zbuf, acc_sh.at[pl.ds(s * ROWS_PER_TILE + k * 64, 64)])
        plsc.subcore_barrier()

        base_e = s * PER_TILE
        for blk in range(NBLK):
            off = pl.multiple_of(base_e + blk * TB, 512)
            pltpu.sync_copy(src_hbm.at[pl.ds(off, TB)], src_v)
            pltpu.sync_copy(dst_hbm.at[pl.ds(off, TB)], dst_v)
            pltpu.sync_copy(
                dst2_hbm.at[pl.ds(pl.multiple_of(off // 64, 8), NSUP)],
                dst2_v)
            pltpu.sync_copy(
                ex_hbm.at[pl.ds(pl.multiple_of(head * EPAD + off, 512),
                                TB)], ex_v)

            def super_body(q, _):
                for j in range(4):
                    didx = dst_v[pl.ds(q * 64 + j * 16, 16)]
                    sidx = src_v[pl.ds(q * 64 + j * 16, 16)]
                    gidx_v[pl.ds(j * 16, 16)] = sidx + g * NPAD
                    rd = plsc.load_gather(rden_v, [didx])
                    abuf[pl.ds(j * 16, 16)] = (
                        ex_v[pl.ds(q * 64 + j * 16, 16)] * rd)
                pltpu.async_copy(xp_hbm.at[gidx_v], stage_v, gsem).wait()
                def row_body(r, _):
                    arep = plsc.load_gather(abuf, [jnp.full((16,), r, _i32)])
                    for cc in range(GW // 16):
                        stage_v[r, pl.ds(cc * 16, 16)] = (
                            stage_v[r, pl.ds(cc * 16, 16)] * arep)
                    return 0
                lax.fori_loop(0, 64, row_body, 0)
                pltpu.sync_copy(stage_v, acc_sh.at[dst2_v.at[q]], add=True)
                return 0

            lax.fori_loop(0, NSUP, super_body, 0)

        plsc.subcore_barrier()
        pltpu.sync_copy(
            acc_sh.at[pl.ds(s * ROWS_PER_TILE, ROWS_PER_TILE)],
            out_hbm.at[g, pl.ds(s * ROWS_PER_TILE, ROWS_PER_TILE)])


def _stage_c(xp_flat, src, dst, dst2, ex_t, rden_t):
    mesh = plsc.VectorSubcoreMesh(core_axis_name="c", subcore_axis_name="s")
    return pl.kernel(
        _stage_c_body,
        out_type=jax.ShapeDtypeStruct((G, NPAD, GW), _f32),
        mesh=mesh,
        compiler_params=pltpu.CompilerParams(needs_layout_passes=False),
        scratch_types=[
            pltpu.VMEM((NPAD,), _f32),
            pltpu.VMEM((TB,), _i32),
            pltpu.VMEM((TB,), _i32),
            pltpu.VMEM((NSUP, 64), _i32),
            pltpu.VMEM((TB,), _f32),
            pltpu.VMEM((64,), _i32),
            pltpu.VMEM((64,), _f32),
            pltpu.VMEM((64, GW), _f32),
            pltpu.VMEM((64, GW), _f32),
            pltpu.MemorySpace.VMEM_SHARED((NPAD, GW), _f32),
            pltpu.SemaphoreType.DMA,
        ],
    )(xp_flat, src, dst, dst2, ex_t, rden_t)


def _stage_d_body(msg_ref, bc_ref, w1_ref, b1_ref, w2_ref, b2_ref,
                  gam_ref, bet_ref, out_ref):
    h1 = jnp.dot(bc_ref[...], w1_ref[...], preferred_element_type=_f32)
    h1 = h1 + b1_ref[...]
    for g in range(G):
        h1 = h1 + jnp.dot(msg_ref[g], w1_ref[g * GW:(g + 1) * GW, :],
                          preferred_element_type=_f32)
    h1 = 0.5 * h1 * (1.0 + lax.erf(h1 * jnp.float32(0.7071067811865476)))
    h2 = jnp.dot(h1, w2_ref[...], preferred_element_type=_f32) + b2_ref[...]
    mu = jnp.mean(h2, axis=-1, keepdims=True)
    d = h2 - mu
    var = jnp.mean(d * d, axis=-1, keepdims=True)
    y = d / jnp.sqrt(var + jnp.float32(1e-5))
    out_ref[...] = y * gam_ref[...] + bet_ref[...]


def _stage_d(msg, b_conv, W1, b1, W2, b2, gamma, beta):
    blk = 512
    grid = NPAD // blk
    return pl.pallas_call(
        _stage_d_body,
        grid=(grid,),
        in_specs=[
            pl.BlockSpec((G, blk, GW), lambda i: (0, i, 0)),
            pl.BlockSpec((1, HC), lambda i: (0, 0)),
            pl.BlockSpec((HC, DFF), lambda i: (0, 0)),
            pl.BlockSpec((1, DFF), lambda i: (0, 0)),
            pl.BlockSpec((DFF, C), lambda i: (0, 0)),
            pl.BlockSpec((1, C), lambda i: (0, 0)),
            pl.BlockSpec((1, C), lambda i: (0, 0)),
            pl.BlockSpec((1, C), lambda i: (0, 0)),
        ],
        out_specs=pl.BlockSpec((blk, C), lambda i: (i, 0)),
        out_shape=jax.ShapeDtypeStruct((NPAD, C), _f32),
    )(msg, b_conv.reshape(1, HC), W1, b1.reshape(1, DFF), W2,
      b2.reshape(1, C), gamma.reshape(1, C), beta.reshape(1, C))


def kernel(x, edge_index, W, att_src, att_dst, b_conv, W1, b1, W2, b2,
           gamma, beta):
    xpad = jnp.pad(x, ((0, NPAD - N), (0, 0)))
    eye = jnp.eye(H, dtype=_f32)
    s_src = (eye[:, None, :] * att_src[:, :, None]).reshape(HC, H)
    s_dst = (eye[:, None, :] * att_dst[:, :, None]).reshape(HC, H)
    S = jnp.concatenate([s_src, s_dst], axis=1)

    loop = jnp.arange(N, dtype=_i32)
    src = jnp.concatenate([edge_index[0].astype(_i32), loop])
    dst = jnp.concatenate([edge_index[1].astype(_i32), loop])
    src = jnp.pad(src, (0, EPAD - E2), constant_values=NPAD - 1)
    dst = jnp.pad(dst, (0, EPAD - E2), constant_values=NPAD - 1)
    dst2 = dst.reshape(EPAD // 64, 64)

    xp_t, aall = _stage_a(xpad, W, S)
    xp_flat = xp_t.reshape(G * NPAD, GW)
    aall_flat = aall.reshape(NPAD * 8)

    ex_t, rden_t = _stage_b(aall_flat, src, dst, dst2)
    msg = _stage_c(xp_flat, src, dst, dst2, ex_t, rden_t)
    y = _stage_d(msg, b_conv, W1, b1, W2, b2, gamma, beta)
    return y[:N]
